# baseline (speedup 1.0000x reference)
import functools

import jax
import jax.numpy as jnp
from jax import lax
from jax.experimental import pallas as pl
from jax.experimental.pallas import tpu as pltpu

_S = 64.0
_M = 0.4

_BR = 16


def _body(lab_ref, x_ref, o_ref):
    x = x_ref[...]
    lab = lab_ref[...]
    col = lax.broadcasted_iota(jnp.int32, x.shape, 1)
    margin = jnp.where(col == lab, jnp.float32(_S * _M), jnp.float32(0.0))
    o_ref[...] = x * jnp.float32(_S) - margin


@functools.cache
def _cosface(rows, vocab):
    return pl.pallas_call(
        _body,
        out_shape=jax.ShapeDtypeStruct((rows, vocab), jnp.float32),
        grid=(rows // _BR,),
        in_specs=[
            pl.BlockSpec((_BR, 1), lambda i: (i, 0)),
            pl.BlockSpec((_BR, vocab), lambda i: (i, 0)),
        ],
        out_specs=pl.BlockSpec((_BR, vocab), lambda i: (i, 0)),
        compiler_params=pltpu.CompilerParams(
            dimension_semantics=("arbitrary",),
        ),
    )


def kernel(logits, labels):
    rows, vocab = logits.shape
    lab2 = labels.astype(jnp.int32).reshape(rows, 1)
    return _cosface(rows, vocab)(lab2, logits)
